# Initial kernel scaffold; baseline (speedup 1.0000x reference)
import jax
import jax.numpy as jnp
from jax.experimental import pallas as pl


def kernel(x, mask):
    raise NotImplementedError("write your pallas kernel here")


# speedup vs baseline: 1.0167x; 1.0167x over previous
import jax
import jax.numpy as jnp
from jax.experimental import pallas as pl
from jax.experimental.pallas import tpu as pltpu

_ROWS = 512


def _body(m_ref, x_ref, o_ref):
    keep = jax.nn.sigmoid(m_ref[...]) >= 0.5
    x = x_ref[...]
    o_ref[...] = jnp.where(keep[None], jnp.maximum(x, 0.0), x)


def kernel(x, mask):
    b, c, h, w = x.shape
    n = c * h
    xr = x.reshape(b, n, w)
    mr = mask.reshape(n, w)
    out = pl.pallas_call(
        _body,
        grid=(n // _ROWS,),
        in_specs=[
            pl.BlockSpec((_ROWS, w), lambda i: (i, 0)),
            pl.BlockSpec((b, _ROWS, w), lambda i: (0, i, 0)),
        ],
        out_specs=pl.BlockSpec((b, _ROWS, w), lambda i: (0, i, 0)),
        out_shape=jax.ShapeDtypeStruct((b, n, w), x.dtype),
        compiler_params=pltpu.CompilerParams(dimension_semantics=("arbitrary",)),
    )(mr, xr)
    return out.reshape(x.shape)


# speedup vs baseline: 1.0170x; 1.0003x over previous
import jax
import jax.numpy as jnp
from jax.experimental import pallas as pl
from jax.experimental.pallas import tpu as pltpu

_ROWS = 512


def _body(m_ref, x_ref, o_ref):
    keep = jax.nn.sigmoid(m_ref[...]) >= 0.5
    x = x_ref[...]
    o_ref[...] = jnp.where(keep[None], jnp.maximum(x, 0.0), x)


def kernel(x, mask):
    b, c, h, w = x.shape
    n = c * h
    xr = x.reshape(b, n, w)
    mr = mask.reshape(n, w)
    out = pl.pallas_call(
        _body,
        grid=(n // _ROWS,),
        in_specs=[
            pl.BlockSpec((_ROWS, w), lambda i: (i, 0)),
            pl.BlockSpec((b, _ROWS, w), lambda i: (0, i, 0)),
        ],
        out_specs=pl.BlockSpec((b, _ROWS, w), lambda i: (0, i, 0)),
        out_shape=jax.ShapeDtypeStruct((b, n, w), x.dtype),
        compiler_params=pltpu.CompilerParams(dimension_semantics=("parallel",)),
    )(mr, xr)
    return out.reshape(x.shape)


# speedup vs baseline: 1.0244x; 1.0072x over previous
import jax
import jax.numpy as jnp
from jax.experimental import pallas as pl
from jax.experimental.pallas import tpu as pltpu

_ROWS = 1024


def _body(m_ref, x_ref, o_ref):
    keep = jax.nn.sigmoid(m_ref[...]) >= 0.5
    x = x_ref[...]
    o_ref[...] = jnp.where(keep[None], jnp.maximum(x, 0.0), x)


def kernel(x, mask):
    b, c, h, w = x.shape
    n = c * h
    xr = x.reshape(b, n, w)
    mr = mask.reshape(n, w)
    out = pl.pallas_call(
        _body,
        grid=(n // _ROWS,),
        in_specs=[
            pl.BlockSpec((_ROWS, w), lambda i: (i, 0)),
            pl.BlockSpec((b, _ROWS, w), lambda i: (0, i, 0)),
        ],
        out_specs=pl.BlockSpec((b, _ROWS, w), lambda i: (0, i, 0)),
        out_shape=jax.ShapeDtypeStruct((b, n, w), x.dtype),
        compiler_params=pltpu.CompilerParams(dimension_semantics=("parallel",)),
    )(mr, xr)
    return out.reshape(x.shape)
